# baseline (speedup 1.0000x reference)
import functools

import jax
import jax.numpy as jnp
from jax import lax
from jax.experimental import pallas as pl
from jax.experimental.pallas import tpu as pltpu
from jax.experimental.pallas import tpu_sc as plsc

H = 16
T = 961
N = 256
HH = 4
ROWS = 32
GROUPS = ROWS * N // 16

_mesh = plsc.VectorSubcoreMesh(core_axis_name="c", subcore_axis_name="s")


@functools.partial(
    pl.kernel,
    mesh=_mesh,
    out_type=jax.ShapeDtypeStruct((H, N, N), jnp.float32),
    scratch_types=[
        pltpu.VMEM((HH, T), jnp.float32),
        pltpu.VMEM((ROWS, N), jnp.int32),
        pltpu.VMEM((HH, ROWS, N), jnp.float32),
        pltpu.SemaphoreType.DMA,
        pltpu.SemaphoreType.DMA,
    ],
    compiler_params=pltpu.CompilerParams(
        needs_layout_passes=False,
        disable_bounds_checks=True,
        disable_semaphore_checks=True,
    ),
)
def _bias_kernel(tab_hbm, idx_hbm, out_hbm, tab_v, idx_v, out_v, sem_t, sem_i):
    wid = lax.axis_index("s") * 2 + lax.axis_index("c")
    h0 = (wid & 3) * HH
    row0 = (wid >> 2) * ROWS
    cp_t = pltpu.async_copy(tab_hbm.at[pl.ds(h0, HH), :], tab_v, sem_t)
    cp_i = pltpu.async_copy(idx_hbm.at[pl.ds(row0, ROWS), :], idx_v, sem_i)
    cp_t.wait()
    cp_i.wait()

    @plsc.parallel_loop(0, GROUPS, unroll=1)
    def body(g):
        r = g >> 4
        c = (g & 15) * 16
        iv = idx_v[r, pl.ds(c, 16)]
        for h in range(HH):
            hv = jnp.full((16,), h, dtype=jnp.int32)
            out_v[h, r, pl.ds(c, 16)] = plsc.load_gather(tab_v, [hv, iv])

    pltpu.sync_copy(out_v, out_hbm.at[pl.ds(h0, HH), pl.ds(row0, ROWS), :])


def kernel(table, index):
    tab_t = jnp.transpose(table)
    out = _bias_kernel(tab_t, index.astype(jnp.int32))
    return out.reshape(1, H, N, N)


# speedup vs baseline: 1.0489x; 1.0489x over previous
import functools

import jax
import jax.numpy as jnp
from jax import lax
from jax.experimental import pallas as pl
from jax.experimental.pallas import tpu as pltpu
from jax.experimental.pallas import tpu_sc as plsc

H = 16
T = 961
N = 256
HH = 8
ROWS = 16
GROUPS = ROWS * N // 16

_mesh = plsc.VectorSubcoreMesh(core_axis_name="c", subcore_axis_name="s")


@functools.partial(
    pl.kernel,
    mesh=_mesh,
    out_type=jax.ShapeDtypeStruct((H, N, N), jnp.float32),
    scratch_types=[
        pltpu.VMEM((HH, T), jnp.float32),
        pltpu.VMEM((ROWS, N), jnp.int32),
        pltpu.VMEM((HH, ROWS, N), jnp.float32),
        pltpu.SemaphoreType.DMA,
        pltpu.SemaphoreType.DMA,
    ],
    compiler_params=pltpu.CompilerParams(
        needs_layout_passes=False,
        disable_bounds_checks=True,
        disable_semaphore_checks=True,
    ),
)
def _bias_kernel(tab_hbm, idx_hbm, out_hbm, tab_v, idx_v, out_v, sem_t, sem_i):
    wid = lax.axis_index("s") * 2 + lax.axis_index("c")
    h0 = (wid & 1) * HH
    row0 = (wid >> 1) * ROWS
    cp_t = pltpu.async_copy(tab_hbm.at[pl.ds(h0, HH), :], tab_v, sem_t)
    cp_i = pltpu.async_copy(idx_hbm.at[pl.ds(row0, ROWS), :], idx_v, sem_i)
    cp_t.wait()
    cp_i.wait()

    @plsc.parallel_loop(0, GROUPS, unroll=2)
    def body(g):
        r = g >> 4
        c = (g & 15) * 16
        iv = idx_v[r, pl.ds(c, 16)]
        for h in range(HH):
            hv = jnp.full((16,), h, dtype=jnp.int32)
            out_v[h, r, pl.ds(c, 16)] = plsc.load_gather(tab_v, [hv, iv])

    pltpu.sync_copy(out_v, out_hbm.at[pl.ds(h0, HH), pl.ds(row0, ROWS), :])


def kernel(table, index):
    tab_t = jnp.transpose(table)
    out = _bias_kernel(tab_t, index.astype(jnp.int32))
    return out.reshape(1, H, N, N)
